# baseline (speedup 1.0000x reference)
import functools

import jax
import jax.numpy as jnp
from jax import lax
from jax.experimental import pallas as pl
from jax.experimental.pallas import tpu as pltpu
from jax.experimental.pallas import tpu_sc as plsc

N_ATOMS = 10000
N_PAIRS = 160000
NPAD = 10240
IN_F = 128
HID = 16
N_RBF = 43
N_HEADS = 7
N_COEF = 112
SPLIT = 128000

_EB = 2000
_NB = 1000


def _f32(*shape):
    return jax.ShapeDtypeStruct(shape, jnp.float32)


def _stage_a_body(q_ref, w_ref, tab_ref):
    tab_ref[...] = jnp.dot(q_ref[...], w_ref[...],
                           preferred_element_type=jnp.float32)


def _stage_a(q, WALL, *, interpret=False):
    grid = N_ATOMS // _NB
    return pl.pallas_call(
        _stage_a_body,
        grid=(grid,),
        in_specs=[
            pl.BlockSpec((_NB, IN_F), lambda i: (i, 0)),
            pl.BlockSpec((IN_F, 128), lambda i: (0, 0)),
        ],
        out_specs=pl.BlockSpec((_NB, 128), lambda i: (i, 0)),
        out_shape=_f32(N_ATOMS, 128),
        interpret=interpret,
    )(q, WALL)


def _stage_b(tab, idx_i, idx_j, e_lo, n_edges):
    mesh = plsc.VectorSubcoreMesh(core_axis_name="c", subcore_axis_name="s")
    nw = mesh.num_cores * mesh.num_subcores
    per_w = n_edges // nw
    chunk = 200
    n_chunks = per_w // chunk

    @functools.partial(
        pl.kernel,
        out_type=[_f32(n_edges, 128)] * 2,
        mesh=mesh,
        scratch_types=[
            pltpu.VMEM((chunk,), jnp.int32),
            pltpu.VMEM((chunk, 128), jnp.float32),
            pltpu.SemaphoreType.DMA,
        ],
    )
    def k(tab_hbm, ii_hbm, ij_hbm, oi_hbm, oj_hbm, idx_v, rows_v, sem):
        wid = lax.axis_index("s") * mesh.num_cores + lax.axis_index("c")
        base = wid * per_w
        for idx_hbm, out_hbm in ((ii_hbm, oi_hbm), (ij_hbm, oj_hbm)):
            for j in range(n_chunks):
                e0 = base + j * chunk
                pltpu.sync_copy(idx_hbm.at[pl.ds(e_lo + e0, chunk)], idx_v)
                pltpu.async_copy(tab_hbm.at[idx_v], rows_v, sem).wait()
                pltpu.sync_copy(rows_v, out_hbm.at[pl.ds(e0, chunk), :])

    return k(tab, idx_i, idx_j)


def _celu2(x):
    return jnp.where(x > 0, x, 2.0 * (jnp.exp(x * 0.5) - 1.0))


def _stage_c_body(egi_ref, egj_ref, dr_ref, m1i_ref, m1j_ref, m2i_ref, m2j_ref,
                  w43_ref, wd_ref, bin_ref, bo1_ref, wo2_ref, bo2_ref,
                  wsem_ref, bsem_ref, ncoef_ref, off_ref, p16_ref, p8_ref,
                  p3_ref, qe_ref):
    egi = egi_ref[...]
    egj = egj_ref[...]
    dr = dr_ref[...]
    d = dr[:, 0:1]
    q_in = (jnp.dot(egi, m1i_ref[...], preferred_element_type=jnp.float32)
            + jnp.dot(egj, m1j_ref[...], preferred_element_type=jnp.float32)
            + bin_ref[...])
    rbf = jnp.exp(ncoef_ref[...] * (d - off_ref[...]) ** 2)
    q_filt = rbf * q_in
    hpre = (jnp.dot(egi, m2i_ref[...], preferred_element_type=jnp.float32)
            + jnp.dot(egj, m2j_ref[...], preferred_element_type=jnp.float32)
            + jnp.dot(q_filt, w43_ref[...], preferred_element_type=jnp.float32)
            + d * wd_ref[...] + bo1_ref[...])
    h = jax.nn.silu(hpre)
    q16 = jnp.dot(h, wo2_ref[...], preferred_element_type=jnp.float32) + bo2_ref[...]
    att8 = jnp.dot(q16, wsem_ref[...], preferred_element_type=jnp.float32) + bsem_ref[...]
    e8 = jnp.exp(_celu2(att8))
    rhat = dr[:, 1:4] * (1.0 / (d + 1e-05))
    qe_ref[...] = (jnp.dot(q16, p16_ref[...], preferred_element_type=jnp.float32)
                   + jnp.dot(e8, p8_ref[...], preferred_element_type=jnp.float32)
                   + jnp.dot(rhat, p3_ref[...], preferred_element_type=jnp.float32))


def _stage_c(egi, egj, dr, M1i, M1j, M2i, M2j, W43, wd, b_in2, b_o12, W_o2,
             b_o22, Wsem8, bsem8, ncoef2, off2, P16, P8, P3, e_lo, n_edges,
             *, interpret=False):
    grid = n_edges // _EB
    off = e_lo // _EB
    full = lambda a, b: pl.BlockSpec((a, b), lambda i: (0, 0))
    return pl.pallas_call(
        _stage_c_body,
        grid=(grid,),
        in_specs=[
            pl.BlockSpec((_EB, 128), lambda i: (i, 0)),
            pl.BlockSpec((_EB, 128), lambda i: (i, 0)),
            pl.BlockSpec((_EB, 4), lambda i: (i + off, 0)),
            full(128, N_RBF), full(128, N_RBF), full(128, HID), full(128, HID),
            full(N_RBF, HID), full(1, HID),
            full(1, N_RBF), full(1, HID), full(HID, HID), full(1, HID),
            full(HID, 8), full(1, 8), full(1, N_RBF), full(1, N_RBF),
            full(HID, 128), full(8, 128), full(3, 128),
        ],
        out_specs=pl.BlockSpec((_EB, 128), lambda i: (i, 0)),
        out_shape=_f32(n_edges, 128),
        interpret=interpret,
    )(egi, egj, dr, M1i, M1j, M2i, M2j, W43, wd, b_in2, b_o12, W_o2, b_o22,
      Wsem8, bsem8, ncoef2, off2, P16, P8, P3)


def _stage_d(qe1, qe2, idx_j, zeros_pad):
    mesh = plsc.VectorSubcoreMesh(core_axis_name="c", subcore_axis_name="s")
    ns = mesh.num_subcores
    chunk = 200
    half1 = SPLIT // 2
    half2 = (N_PAIRS - SPLIT) // 2
    per_t1 = half1 // ns
    per_t2 = half2 // ns
    stripe = NPAD // ns

    @functools.partial(
        pl.kernel,
        out_type=[_f32(NPAD, 128)] * 2,
        mesh=mesh,
        scratch_types=[
            pltpu.VMEM((chunk,), jnp.int32),
            pltpu.VMEM((chunk, 128), jnp.float32),
            pltpu.VMEM_SHARED((NPAD, 128), jnp.float32),
        ],
    )
    def k(q1_hbm, q2_hbm, ij_hbm, z_hbm, s0_hbm, s1_hbm, idx_v, upd_v, acc):
        cid = lax.axis_index("c")
        sid = lax.axis_index("s")
        r0 = sid * stripe
        pltpu.sync_copy(z_hbm.at[pl.ds(r0, stripe), :], acc.at[pl.ds(r0, stripe)])
        plsc.subcore_barrier()
        for qe_hbm, g_base, half, per_t in ((q1_hbm, 0, half1, per_t1),
                                            (q2_hbm, SPLIT, half2, per_t2)):
            for j in range(per_t // chunk):
                e0 = cid * half + sid * per_t + j * chunk
                pltpu.sync_copy(ij_hbm.at[pl.ds(g_base + e0, chunk)], idx_v)
                pltpu.sync_copy(qe_hbm.at[pl.ds(e0, chunk), :], upd_v)
                pltpu.sync_copy(upd_v, acc.at[idx_v], add=True)
        plsc.subcore_barrier()

        @pl.when(cid == 0)
        def _():
            pltpu.sync_copy(acc.at[pl.ds(r0, stripe)], s0_hbm.at[pl.ds(r0, stripe)])

        @pl.when(cid == 1)
        def _():
            pltpu.sync_copy(acc.at[pl.ds(r0, stripe)], s1_hbm.at[pl.ds(r0, stripe)])

    return k(qe1, qe2, idx_j, zeros_pad)


def _stage_f(s8, idx_j):
    mesh = plsc.VectorSubcoreMesh(core_axis_name="c", subcore_axis_name="s")
    nw = mesh.num_cores * mesh.num_subcores
    per_w = N_PAIRS // nw
    chunk = 1000
    n_chunks = per_w // chunk

    @functools.partial(
        pl.kernel,
        out_type=_f32(N_PAIRS, 8),
        mesh=mesh,
        compiler_params=pltpu.CompilerParams(use_tc_tiling_on_sc=False),
        scratch_types=[
            pltpu.VMEM((chunk,), jnp.int32),
            pltpu.VMEM((chunk, 8), jnp.float32),
            pltpu.SemaphoreType.DMA,
        ],
    )
    def k(s_hbm, ij_hbm, out_hbm, idx_v, rows_v, sem):
        wid = lax.axis_index("s") * mesh.num_cores + lax.axis_index("c")
        base = wid * per_w
        for j in range(n_chunks):
            e0 = base + j * chunk
            pltpu.sync_copy(ij_hbm.at[pl.ds(e0, chunk)], idx_v)
            pltpu.async_copy(s_hbm.at[idx_v], rows_v, sem).wait()
            pltpu.sync_copy(rows_v, out_hbm.at[pl.ds(e0, chunk), :])

    return k(s8, idx_j)


def _stage_g_body(qe_ref, sg_ref, u16_ref, u8_ref, rep16_ref, rep7_ref,
                  wmix_ref, p0_ref, p1_ref, p2_ref, p3_ref):
    qe = qe_ref[...]
    q16 = jnp.dot(qe, u16_ref[...], preferred_element_type=jnp.float32)
    e8 = jnp.dot(qe, u8_ref[...], preferred_element_type=jnp.float32)
    comb = e8 / sg_ref[...]
    qr = jnp.dot(q16, rep16_ref[...], preferred_element_type=jnp.float32)
    cr = jnp.dot(comb, rep7_ref[...], preferred_element_type=jnp.float32)
    qia = qr * cr
    co = jnp.tanh(jnp.dot(qia, wmix_ref[...], preferred_element_type=jnp.float32))
    p0_ref[...] = qia
    p1_ref[...] = qe[:, 24:25] * co
    p2_ref[...] = qe[:, 25:26] * co
    p3_ref[...] = qe[:, 26:27] * co


def _stage_g(qe, sg, U16, U8, Rep16, Rep7, W_mix128,
             e_lo, n_edges, *, interpret=False):
    grid = n_edges // _EB
    off = e_lo // _EB
    full = lambda a, b: pl.BlockSpec((a, b), lambda i: (0, 0))
    loc = lambda w: pl.BlockSpec((_EB, w), lambda i: (i, 0))
    glo = lambda w: pl.BlockSpec((_EB, w), lambda i: (i + off, 0))
    return pl.pallas_call(
        _stage_g_body,
        grid=(grid,),
        in_specs=[
            loc(128), glo(8),
            full(128, HID), full(128, 8),
            full(HID, 128), full(8, 128), full(128, 128),
        ],
        out_specs=[loc(128)] * 4,
        out_shape=[_f32(n_edges, 128)] * 4,
        interpret=interpret,
    )(qe, sg, U16, U8, Rep16, Rep7, W_mix128)


def _stage_h(pay0, pay1, pay2, pay3, idx_j, zeros_pad, e_lo, n_edges):
    mesh = plsc.VectorSubcoreMesh(core_axis_name="c", subcore_axis_name="s")
    ns = mesh.num_subcores
    per_t = n_edges // ns
    chunk = 200
    n_chunks = per_t // chunk
    stripe = NPAD // ns

    @functools.partial(
        pl.kernel,
        out_type=[_f32(NPAD, 128)] * 4,
        mesh=mesh,
        scratch_types=[
            pltpu.VMEM((chunk,), jnp.int32),
            pltpu.VMEM((chunk, 128), jnp.float32),
            pltpu.VMEM_SHARED((NPAD, 128), jnp.float32),
        ],
    )
    def k(p0_hbm, p1_hbm, p2_hbm, p3_hbm, ij_hbm, z_hbm,
          o0_hbm, o1_hbm, o2_hbm, o3_hbm, idx_v, upd_v, acc):
        cid = lax.axis_index("c")
        sid = lax.axis_index("s")
        r0 = sid * stripe

        def run_group(pay_hbm, out_hbm):
            pltpu.sync_copy(z_hbm.at[pl.ds(r0, stripe), :], acc.at[pl.ds(r0, stripe)])
            plsc.subcore_barrier()
            for j in range(n_chunks):
                e0 = sid * per_t + j * chunk
                pltpu.sync_copy(ij_hbm.at[pl.ds(e_lo + e0, chunk)], idx_v)
                pltpu.sync_copy(pay_hbm.at[pl.ds(e0, chunk), :], upd_v)
                pltpu.sync_copy(upd_v, acc.at[idx_v], add=True)
            plsc.subcore_barrier()
            pltpu.sync_copy(acc.at[pl.ds(r0, stripe)], out_hbm.at[pl.ds(r0, stripe)])
            plsc.subcore_barrier()

        @pl.when(cid == 0)
        def _():
            run_group(p0_hbm, o0_hbm)
            run_group(p1_hbm, o1_hbm)

        @pl.when(cid == 1)
        def _():
            run_group(p2_hbm, o2_hbm)
            run_group(p3_hbm, o3_hbm)

    return k(pay0, pay1, pay2, pay3, idx_j, zeros_pad)


def _stage_i_body(q_ref, g0a_ref, g1a_ref, g2a_ref, g3a_ref,
                  g0b_ref, g1b_ref, g2b_ref, g3b_ref, s_ref,
                  wp1_ref, bp1_ref, wp2_ref, bp2_ref,
                  wn1q_ref, wn1ij_ref, wn1c_ref, bn1_ref, wn2_ref, bn2_ref,
                  out_ref):
    q = q_ref[...]
    qij = g0a_ref[...] + g0b_ref[...]
    inv = 1.0 / jnp.maximum(s_ref[...][:, 7:8], 1.0)
    m1 = (g1a_ref[...] + g1b_ref[...]) * inv
    m2 = (g2a_ref[...] + g2b_ref[...]) * inv
    m3 = (g3a_ref[...] + g3b_ref[...]) * inv
    comb_norm = m1 * m1 + m2 * m2 + m3 * m3
    qc = jax.nn.silu(jnp.dot(comb_norm, wp1_ref[...],
                             preferred_element_type=jnp.float32) + bp1_ref[...])
    qc = jax.nn.silu(jnp.dot(qc, wp2_ref[...],
                             preferred_element_type=jnp.float32) + bp2_ref[...])
    o = (jnp.dot(q, wn1q_ref[...], preferred_element_type=jnp.float32)
         + jnp.dot(qij, wn1ij_ref[...], preferred_element_type=jnp.float32)
         + jnp.dot(qc, wn1c_ref[...], preferred_element_type=jnp.float32)
         + bn1_ref[...])
    o = jax.nn.silu(o)
    o = jax.nn.silu(jnp.dot(o, wn2_ref[...], preferred_element_type=jnp.float32)
                    + bn2_ref[...])
    out_ref[...] = q + o


def _stage_i(q, ga, gb, s8, W_p1p, b_p12, W_p2, b_p22,
             Wn1q, Wn1ij, Wn1c, b_n12, W_n2, b_n22, *, interpret=False):
    grid = N_ATOMS // _NB
    full = lambda a, b: pl.BlockSpec((a, b), lambda i: (0, 0))
    nb = lambda w: pl.BlockSpec((_NB, w), lambda i: (i, 0))
    return pl.pallas_call(
        _stage_i_body,
        grid=(grid,),
        in_specs=[
            nb(IN_F), nb(128), nb(128), nb(128), nb(128),
            nb(128), nb(128), nb(128), nb(128), nb(8),
            full(128, HID), full(1, HID), full(HID, HID), full(1, HID),
            full(IN_F, HID), full(128, HID), full(HID, HID), full(1, HID),
            full(HID, IN_F), full(1, IN_F),
        ],
        out_specs=nb(IN_F),
        out_shape=_f32(N_ATOMS, IN_F),
        interpret=interpret,
    )(q, ga[0], ga[1], ga[2], ga[3], gb[0], gb[1], gb[2], gb[3], s8,
      W_p1p, b_p12, W_p2, b_p22, Wn1q, Wn1ij, Wn1c, b_n12, W_n2, b_n22)


def kernel(q, mu, r_ij, d_ij, idx_i, idx_j, rbf_offsets, rbf_widths, W_in, b_in,
           W_o1, b_o1, W_o2, b_o2, W_sem, b_sem, W_mix, W_p1, b_p1, W_p2, b_p2,
           W_n1, b_n1, W_n2, b_n2):
    f32 = jnp.float32
    z5 = jnp.zeros((IN_F, 5), f32)
    WALL = jnp.concatenate([W_in[:IN_F], z5, W_o1[:IN_F],
                            W_in[IN_F:], z5, W_o1[IN_F:2 * IN_F]], axis=1)
    eye43 = jnp.eye(N_RBF, dtype=f32)
    eye16 = jnp.eye(HID, dtype=f32)
    eye8 = jnp.eye(8, dtype=f32)
    M1i = jnp.concatenate([eye43, jnp.zeros((85, N_RBF), f32)], axis=0)
    M1j = jnp.concatenate([jnp.zeros((64, N_RBF), f32), eye43,
                           jnp.zeros((21, N_RBF), f32)], axis=0)
    M2i = jnp.concatenate([jnp.zeros((48, HID), f32), eye16,
                           jnp.zeros((64, HID), f32)], axis=0)
    M2j = jnp.concatenate([jnp.zeros((112, HID), f32), eye16], axis=0)
    W43 = W_o1[2 * IN_F:2 * IN_F + N_RBF]
    wd = W_o1[2 * IN_F + N_RBF:2 * IN_F + N_RBF + 1]
    Wsem8 = jnp.concatenate([W_sem, jnp.zeros((HID, 1), f32)], axis=1)
    bsem8 = jnp.concatenate([b_sem, jnp.zeros((1,), f32)])[None, :]
    ncoef2 = (-0.5 / (rbf_widths ** 2))[None, :]
    off2 = rbf_offsets[None, :]
    P16 = jnp.concatenate([eye16, jnp.zeros((HID, 112), f32)], axis=1)
    P8 = jnp.concatenate([jnp.zeros((8, 16), f32), eye8,
                          jnp.zeros((8, 104), f32)], axis=1)
    U16 = P16.T
    U8 = P8.T
    ar = jnp.arange(128)
    Rep16 = ((ar[None, :] // N_HEADS == jnp.arange(HID)[:, None])
             & (ar[None, :] < N_COEF)).astype(f32)
    Rep7 = ((ar[None, :] % N_HEADS == jnp.arange(8)[:, None])
            & (ar[None, :] < N_COEF)).astype(f32)
    W_mix128 = jnp.zeros((128, 128), f32).at[:N_COEF, :N_COEF].set(W_mix)
    P3 = jnp.zeros((3, 128), f32).at[:, 24:27].set(jnp.eye(3, dtype=f32))
    Wn1q = W_n1[:IN_F]
    Wn1ij = jnp.concatenate([W_n1[IN_F:IN_F + N_COEF],
                             jnp.zeros((16, HID), f32)], axis=0)
    Wn1c = W_n1[IN_F + N_COEF:]
    Wp1p = jnp.concatenate([W_p1, jnp.zeros((16, HID), f32)], axis=0)
    zeros_pad = jnp.zeros((NPAD, 128), f32)
    DR = jnp.concatenate([d_ij[:, None], r_ij], axis=1)

    n1, n2 = SPLIT, N_PAIRS - SPLIT

    tab = _stage_a(q, WALL)
    egi1, egj1 = _stage_b(tab, idx_i, idx_j, 0, n1)
    egi2, egj2 = _stage_b(tab, idx_i, idx_j, SPLIT, n2)
    cargs = (M1i, M1j, M2i, M2j, W43, wd, b_in[None, :], b_o1[None, :],
             W_o2, b_o2[None, :], Wsem8, bsem8, ncoef2, off2, P16, P8, P3)
    qe1 = _stage_c(egi1, egj1, DR, *cargs, 0, n1)
    qe2 = _stage_c(egi2, egj2, DR, *cargs, SPLIT, n2)
    s128a, s128b = _stage_d(qe1, qe2, idx_j, zeros_pad)
    s8 = (s128a + s128b)[:, 16:24]
    sg = _stage_f(s8, idx_j)
    pay1 = _stage_g(qe1, sg, U16, U8, Rep16, Rep7, W_mix128, 0, n1)
    pay2 = _stage_g(qe2, sg, U16, U8, Rep16, Rep7, W_mix128, SPLIT, n2)
    ga = _stage_h(pay1[0], pay1[1], pay1[2], pay1[3], idx_j, zeros_pad, 0, n1)
    gb = _stage_h(pay2[0], pay2[1], pay2[2], pay2[3], idx_j, zeros_pad,
                  SPLIT, n2)
    out = _stage_i(q, ga, gb, s8, Wp1p, b_p1[None, :], W_p2,
                   b_p2[None, :], Wn1q, Wn1ij, Wn1c, b_n1[None, :], W_n2,
                   b_n2[None, :])
    return out


# speedup vs baseline: 1.0236x; 1.0236x over previous
import functools

import jax
import jax.numpy as jnp
from jax import lax
from jax.experimental import pallas as pl
from jax.experimental.pallas import tpu as pltpu
from jax.experimental.pallas import tpu_sc as plsc

N_ATOMS = 10000
N_PAIRS = 160000
NPAD = 10240
IN_F = 128
HID = 16
N_RBF = 43
N_HEADS = 7
N_COEF = 112
SPLIT = 96000

_EB = 2000
_NB = 1000


def _f32(*shape):
    return jax.ShapeDtypeStruct(shape, jnp.float32)


def _stage_a_body(q_ref, w_ref, tab_ref):
    tab_ref[...] = jnp.dot(q_ref[...], w_ref[...],
                           preferred_element_type=jnp.float32)


def _stage_a(q, WALL, *, interpret=False):
    grid = N_ATOMS // _NB
    return pl.pallas_call(
        _stage_a_body,
        grid=(grid,),
        in_specs=[
            pl.BlockSpec((_NB, IN_F), lambda i: (i, 0)),
            pl.BlockSpec((IN_F, 128), lambda i: (0, 0)),
        ],
        out_specs=pl.BlockSpec((_NB, 128), lambda i: (i, 0)),
        out_shape=_f32(N_ATOMS, 128),
        interpret=interpret,
    )(q, WALL)


def _stage_b(tab, idx_i, idx_j, e_lo, n_edges):
    mesh = plsc.VectorSubcoreMesh(core_axis_name="c", subcore_axis_name="s")
    nw = mesh.num_cores * mesh.num_subcores
    per_w = n_edges // nw
    chunk = 200
    n_chunks = per_w // chunk

    @functools.partial(
        pl.kernel,
        out_type=[_f32(n_edges, 128)] * 2,
        mesh=mesh,
        scratch_types=[
            pltpu.VMEM((chunk,), jnp.int32),
            pltpu.VMEM((chunk, 128), jnp.float32),
            pltpu.SemaphoreType.DMA,
        ],
    )
    def k(tab_hbm, ii_hbm, ij_hbm, oi_hbm, oj_hbm, idx_v, rows_v, sem):
        wid = lax.axis_index("s") * mesh.num_cores + lax.axis_index("c")
        base = wid * per_w
        for idx_hbm, out_hbm in ((ii_hbm, oi_hbm), (ij_hbm, oj_hbm)):
            for j in range(n_chunks):
                e0 = base + j * chunk
                pltpu.sync_copy(idx_hbm.at[pl.ds(e_lo + e0, chunk)], idx_v)
                pltpu.async_copy(tab_hbm.at[idx_v], rows_v, sem).wait()
                pltpu.sync_copy(rows_v, out_hbm.at[pl.ds(e0, chunk), :])

    return k(tab, idx_i, idx_j)


def _celu2(x):
    return jnp.where(x > 0, x, 2.0 * (jnp.exp(x * 0.5) - 1.0))


def _stage_c_body(egi_ref, egj_ref, dr_ref, m1i_ref, m1j_ref, m2i_ref, m2j_ref,
                  w43_ref, wd_ref, bin_ref, bo1_ref, wo2_ref, bo2_ref,
                  wsem_ref, bsem_ref, ncoef_ref, off_ref, p16_ref, p8_ref,
                  p3_ref, qe_ref):
    egi = egi_ref[...]
    egj = egj_ref[...]
    dr = dr_ref[...]
    d = dr[:, 0:1]
    q_in = (jnp.dot(egi, m1i_ref[...], preferred_element_type=jnp.float32)
            + jnp.dot(egj, m1j_ref[...], preferred_element_type=jnp.float32)
            + bin_ref[...])
    rbf = jnp.exp(ncoef_ref[...] * (d - off_ref[...]) ** 2)
    q_filt = rbf * q_in
    hpre = (jnp.dot(egi, m2i_ref[...], preferred_element_type=jnp.float32)
            + jnp.dot(egj, m2j_ref[...], preferred_element_type=jnp.float32)
            + jnp.dot(q_filt, w43_ref[...], preferred_element_type=jnp.float32)
            + d * wd_ref[...] + bo1_ref[...])
    h = jax.nn.silu(hpre)
    q16 = jnp.dot(h, wo2_ref[...], preferred_element_type=jnp.float32) + bo2_ref[...]
    att8 = jnp.dot(q16, wsem_ref[...], preferred_element_type=jnp.float32) + bsem_ref[...]
    e8 = jnp.exp(_celu2(att8))
    rhat = dr[:, 1:4] * (1.0 / (d + 1e-05))
    qe_ref[...] = (jnp.dot(q16, p16_ref[...], preferred_element_type=jnp.float32)
                   + jnp.dot(e8, p8_ref[...], preferred_element_type=jnp.float32)
                   + jnp.dot(rhat, p3_ref[...], preferred_element_type=jnp.float32))


def _stage_c(egi, egj, dr, M1i, M1j, M2i, M2j, W43, wd, b_in2, b_o12, W_o2,
             b_o22, Wsem8, bsem8, ncoef2, off2, P16, P8, P3, e_lo, n_edges,
             *, interpret=False):
    grid = n_edges // _EB
    off = e_lo // _EB
    full = lambda a, b: pl.BlockSpec((a, b), lambda i: (0, 0))
    return pl.pallas_call(
        _stage_c_body,
        grid=(grid,),
        in_specs=[
            pl.BlockSpec((_EB, 128), lambda i: (i, 0)),
            pl.BlockSpec((_EB, 128), lambda i: (i, 0)),
            pl.BlockSpec((_EB, 4), lambda i: (i + off, 0)),
            full(128, N_RBF), full(128, N_RBF), full(128, HID), full(128, HID),
            full(N_RBF, HID), full(1, HID),
            full(1, N_RBF), full(1, HID), full(HID, HID), full(1, HID),
            full(HID, 8), full(1, 8), full(1, N_RBF), full(1, N_RBF),
            full(HID, 128), full(8, 128), full(3, 128),
        ],
        out_specs=pl.BlockSpec((_EB, 128), lambda i: (i, 0)),
        out_shape=_f32(n_edges, 128),
        interpret=interpret,
    )(egi, egj, dr, M1i, M1j, M2i, M2j, W43, wd, b_in2, b_o12, W_o2, b_o22,
      Wsem8, bsem8, ncoef2, off2, P16, P8, P3)


def _stage_d(qe1, qe2, idx_j, zeros_pad):
    mesh = plsc.VectorSubcoreMesh(core_axis_name="c", subcore_axis_name="s")
    ns = mesh.num_subcores
    chunk = 200
    half1 = SPLIT // 2
    half2 = (N_PAIRS - SPLIT) // 2
    per_t1 = half1 // ns
    per_t2 = half2 // ns
    stripe = NPAD // ns

    @functools.partial(
        pl.kernel,
        out_type=[_f32(NPAD, 128)] * 2,
        mesh=mesh,
        scratch_types=[
            pltpu.VMEM((chunk,), jnp.int32),
            pltpu.VMEM((chunk, 128), jnp.float32),
            pltpu.VMEM_SHARED((NPAD, 128), jnp.float32),
        ],
    )
    def k(q1_hbm, q2_hbm, ij_hbm, z_hbm, s0_hbm, s1_hbm, idx_v, upd_v, acc):
        cid = lax.axis_index("c")
        sid = lax.axis_index("s")
        r0 = sid * stripe
        pltpu.sync_copy(z_hbm.at[pl.ds(r0, stripe), :], acc.at[pl.ds(r0, stripe)])
        plsc.subcore_barrier()
        for qe_hbm, g_base, half, per_t in ((q1_hbm, 0, half1, per_t1),
                                            (q2_hbm, SPLIT, half2, per_t2)):
            for j in range(per_t // chunk):
                e0 = cid * half + sid * per_t + j * chunk
                pltpu.sync_copy(ij_hbm.at[pl.ds(g_base + e0, chunk)], idx_v)
                pltpu.sync_copy(qe_hbm.at[pl.ds(e0, chunk), :], upd_v)
                pltpu.sync_copy(upd_v, acc.at[idx_v], add=True)
        plsc.subcore_barrier()

        @pl.when(cid == 0)
        def _():
            pltpu.sync_copy(acc.at[pl.ds(r0, stripe)], s0_hbm.at[pl.ds(r0, stripe)])

        @pl.when(cid == 1)
        def _():
            pltpu.sync_copy(acc.at[pl.ds(r0, stripe)], s1_hbm.at[pl.ds(r0, stripe)])

    return k(qe1, qe2, idx_j, zeros_pad)


def _stage_f(s8, idx_j):
    mesh = plsc.VectorSubcoreMesh(core_axis_name="c", subcore_axis_name="s")
    nw = mesh.num_cores * mesh.num_subcores
    per_w = N_PAIRS // nw
    chunk = 1000
    n_chunks = per_w // chunk

    @functools.partial(
        pl.kernel,
        out_type=_f32(N_PAIRS, 8),
        mesh=mesh,
        compiler_params=pltpu.CompilerParams(use_tc_tiling_on_sc=False),
        scratch_types=[
            pltpu.VMEM((chunk,), jnp.int32),
            pltpu.VMEM((chunk, 8), jnp.float32),
            pltpu.SemaphoreType.DMA,
        ],
    )
    def k(s_hbm, ij_hbm, out_hbm, idx_v, rows_v, sem):
        wid = lax.axis_index("s") * mesh.num_cores + lax.axis_index("c")
        base = wid * per_w
        for j in range(n_chunks):
            e0 = base + j * chunk
            pltpu.sync_copy(ij_hbm.at[pl.ds(e0, chunk)], idx_v)
            pltpu.async_copy(s_hbm.at[idx_v], rows_v, sem).wait()
            pltpu.sync_copy(rows_v, out_hbm.at[pl.ds(e0, chunk), :])

    return k(s8, idx_j)


def _stage_g_body(qe_ref, sg_ref, u16_ref, u8_ref, rep16_ref, rep7_ref,
                  wmix_ref, p0_ref, p1_ref, p2_ref, p3_ref):
    qe = qe_ref[...]
    q16 = jnp.dot(qe, u16_ref[...], preferred_element_type=jnp.float32)
    e8 = jnp.dot(qe, u8_ref[...], preferred_element_type=jnp.float32)
    comb = e8 / sg_ref[...]
    qr = jnp.dot(q16, rep16_ref[...], preferred_element_type=jnp.float32)
    cr = jnp.dot(comb, rep7_ref[...], preferred_element_type=jnp.float32)
    qia = qr * cr
    co = jnp.tanh(jnp.dot(qia, wmix_ref[...], preferred_element_type=jnp.float32))
    p0_ref[...] = qia
    p1_ref[...] = qe[:, 24:25] * co
    p2_ref[...] = qe[:, 25:26] * co
    p3_ref[...] = qe[:, 26:27] * co


def _stage_g(qe, sg, U16, U8, Rep16, Rep7, W_mix128,
             qe_lo, g_lo, n_edges, *, interpret=False):
    grid = n_edges // _EB
    qoff = qe_lo // _EB
    goff = g_lo // _EB
    full = lambda a, b: pl.BlockSpec((a, b), lambda i: (0, 0))
    loc = lambda w: pl.BlockSpec((_EB, w), lambda i: (i + qoff, 0))
    glo = lambda w: pl.BlockSpec((_EB, w), lambda i: (i + goff, 0))
    return pl.pallas_call(
        _stage_g_body,
        grid=(grid,),
        in_specs=[
            loc(128), glo(8),
            full(128, HID), full(128, 8),
            full(HID, 128), full(8, 128), full(128, 128),
        ],
        out_specs=[pl.BlockSpec((_EB, 128), lambda i: (i, 0))] * 4,
        out_shape=[_f32(n_edges, 128)] * 4,
        interpret=interpret,
    )(qe, sg, U16, U8, Rep16, Rep7, W_mix128)


def _stage_h(pay0, pay1, pay2, pay3, idx_j, zeros_pad, e_lo, n_edges):
    mesh = plsc.VectorSubcoreMesh(core_axis_name="c", subcore_axis_name="s")
    ns = mesh.num_subcores
    per_t = n_edges // ns
    chunk = 200
    n_chunks = per_t // chunk
    stripe = NPAD // ns

    @functools.partial(
        pl.kernel,
        out_type=[_f32(NPAD, 128)] * 4,
        mesh=mesh,
        scratch_types=[
            pltpu.VMEM((chunk,), jnp.int32),
            pltpu.VMEM((chunk, 128), jnp.float32),
            pltpu.VMEM_SHARED((NPAD, 128), jnp.float32),
        ],
    )
    def k(p0_hbm, p1_hbm, p2_hbm, p3_hbm, ij_hbm, z_hbm,
          o0_hbm, o1_hbm, o2_hbm, o3_hbm, idx_v, upd_v, acc):
        cid = lax.axis_index("c")
        sid = lax.axis_index("s")
        r0 = sid * stripe

        def run_group(pay_hbm, out_hbm):
            pltpu.sync_copy(z_hbm.at[pl.ds(r0, stripe), :], acc.at[pl.ds(r0, stripe)])
            plsc.subcore_barrier()
            for j in range(n_chunks):
                e0 = sid * per_t + j * chunk
                pltpu.sync_copy(ij_hbm.at[pl.ds(e_lo + e0, chunk)], idx_v)
                pltpu.sync_copy(pay_hbm.at[pl.ds(e0, chunk), :], upd_v)
                pltpu.sync_copy(upd_v, acc.at[idx_v], add=True)
            plsc.subcore_barrier()
            pltpu.sync_copy(acc.at[pl.ds(r0, stripe)], out_hbm.at[pl.ds(r0, stripe)])
            plsc.subcore_barrier()

        @pl.when(cid == 0)
        def _():
            run_group(p0_hbm, o0_hbm)
            run_group(p1_hbm, o1_hbm)

        @pl.when(cid == 1)
        def _():
            run_group(p2_hbm, o2_hbm)
            run_group(p3_hbm, o3_hbm)

    return k(pay0, pay1, pay2, pay3, idx_j, zeros_pad)


def _stage_i_body(q_ref, g0a_ref, g1a_ref, g2a_ref, g3a_ref,
                  g0b_ref, g1b_ref, g2b_ref, g3b_ref,
                  g0c_ref, g1c_ref, g2c_ref, g3c_ref, s_ref,
                  wp1_ref, bp1_ref, wp2_ref, bp2_ref,
                  wn1q_ref, wn1ij_ref, wn1c_ref, bn1_ref, wn2_ref, bn2_ref,
                  out_ref):
    q = q_ref[...]
    qij = g0a_ref[...] + g0b_ref[...] + g0c_ref[...]
    inv = 1.0 / jnp.maximum(s_ref[...][:, 7:8], 1.0)
    m1 = (g1a_ref[...] + g1b_ref[...] + g1c_ref[...]) * inv
    m2 = (g2a_ref[...] + g2b_ref[...] + g2c_ref[...]) * inv
    m3 = (g3a_ref[...] + g3b_ref[...] + g3c_ref[...]) * inv
    comb_norm = m1 * m1 + m2 * m2 + m3 * m3
    qc = jax.nn.silu(jnp.dot(comb_norm, wp1_ref[...],
                             preferred_element_type=jnp.float32) + bp1_ref[...])
    qc = jax.nn.silu(jnp.dot(qc, wp2_ref[...],
                             preferred_element_type=jnp.float32) + bp2_ref[...])
    o = (jnp.dot(q, wn1q_ref[...], preferred_element_type=jnp.float32)
         + jnp.dot(qij, wn1ij_ref[...], preferred_element_type=jnp.float32)
         + jnp.dot(qc, wn1c_ref[...], preferred_element_type=jnp.float32)
         + bn1_ref[...])
    o = jax.nn.silu(o)
    o = jax.nn.silu(jnp.dot(o, wn2_ref[...], preferred_element_type=jnp.float32)
                    + bn2_ref[...])
    out_ref[...] = q + o


def _stage_i(q, ga, gb, gc, s8, W_p1p, b_p12, W_p2, b_p22,
             Wn1q, Wn1ij, Wn1c, b_n12, W_n2, b_n22, *, interpret=False):
    grid = N_ATOMS // _NB
    full = lambda a, b: pl.BlockSpec((a, b), lambda i: (0, 0))
    nb = lambda w: pl.BlockSpec((_NB, w), lambda i: (i, 0))
    return pl.pallas_call(
        _stage_i_body,
        grid=(grid,),
        in_specs=[
            nb(IN_F), nb(128), nb(128), nb(128), nb(128),
            nb(128), nb(128), nb(128), nb(128),
            nb(128), nb(128), nb(128), nb(128), nb(8),
            full(128, HID), full(1, HID), full(HID, HID), full(1, HID),
            full(IN_F, HID), full(128, HID), full(HID, HID), full(1, HID),
            full(HID, IN_F), full(1, IN_F),
        ],
        out_specs=nb(IN_F),
        out_shape=_f32(N_ATOMS, IN_F),
        interpret=interpret,
    )(q, ga[0], ga[1], ga[2], ga[3], gb[0], gb[1], gb[2], gb[3],
      gc[0], gc[1], gc[2], gc[3], s8,
      W_p1p, b_p12, W_p2, b_p22, Wn1q, Wn1ij, Wn1c, b_n12, W_n2, b_n22)


def kernel(q, mu, r_ij, d_ij, idx_i, idx_j, rbf_offsets, rbf_widths, W_in, b_in,
           W_o1, b_o1, W_o2, b_o2, W_sem, b_sem, W_mix, W_p1, b_p1, W_p2, b_p2,
           W_n1, b_n1, W_n2, b_n2):
    f32 = jnp.float32
    z5 = jnp.zeros((IN_F, 5), f32)
    WALL = jnp.concatenate([W_in[:IN_F], z5, W_o1[:IN_F],
                            W_in[IN_F:], z5, W_o1[IN_F:2 * IN_F]], axis=1)
    eye43 = jnp.eye(N_RBF, dtype=f32)
    eye16 = jnp.eye(HID, dtype=f32)
    eye8 = jnp.eye(8, dtype=f32)
    M1i = jnp.concatenate([eye43, jnp.zeros((85, N_RBF), f32)], axis=0)
    M1j = jnp.concatenate([jnp.zeros((64, N_RBF), f32), eye43,
                           jnp.zeros((21, N_RBF), f32)], axis=0)
    M2i = jnp.concatenate([jnp.zeros((48, HID), f32), eye16,
                           jnp.zeros((64, HID), f32)], axis=0)
    M2j = jnp.concatenate([jnp.zeros((112, HID), f32), eye16], axis=0)
    W43 = W_o1[2 * IN_F:2 * IN_F + N_RBF]
    wd = W_o1[2 * IN_F + N_RBF:2 * IN_F + N_RBF + 1]
    Wsem8 = jnp.concatenate([W_sem, jnp.zeros((HID, 1), f32)], axis=1)
    bsem8 = jnp.concatenate([b_sem, jnp.zeros((1,), f32)])[None, :]
    ncoef2 = (-0.5 / (rbf_widths ** 2))[None, :]
    off2 = rbf_offsets[None, :]
    P16 = jnp.concatenate([eye16, jnp.zeros((HID, 112), f32)], axis=1)
    P8 = jnp.concatenate([jnp.zeros((8, 16), f32), eye8,
                          jnp.zeros((8, 104), f32)], axis=1)
    U16 = P16.T
    U8 = P8.T
    ar = jnp.arange(128)
    Rep16 = ((ar[None, :] // N_HEADS == jnp.arange(HID)[:, None])
             & (ar[None, :] < N_COEF)).astype(f32)
    Rep7 = ((ar[None, :] % N_HEADS == jnp.arange(8)[:, None])
            & (ar[None, :] < N_COEF)).astype(f32)
    W_mix128 = jnp.zeros((128, 128), f32).at[:N_COEF, :N_COEF].set(W_mix)
    P3 = jnp.zeros((3, 128), f32).at[:, 24:27].set(jnp.eye(3, dtype=f32))
    Wn1q = W_n1[:IN_F]
    Wn1ij = jnp.concatenate([W_n1[IN_F:IN_F + N_COEF],
                             jnp.zeros((16, HID), f32)], axis=0)
    Wn1c = W_n1[IN_F + N_COEF:]
    Wp1p = jnp.concatenate([W_p1, jnp.zeros((16, HID), f32)], axis=0)
    zeros_pad = jnp.zeros((NPAD, 128), f32)
    DR = jnp.concatenate([d_ij[:, None], r_ij], axis=1)

    n1, n2 = SPLIT, N_PAIRS - SPLIT

    tab = _stage_a(q, WALL)
    egi1, egj1 = _stage_b(tab, idx_i, idx_j, 0, n1)
    egi2, egj2 = _stage_b(tab, idx_i, idx_j, SPLIT, n2)
    cargs = (M1i, M1j, M2i, M2j, W43, wd, b_in[None, :], b_o1[None, :],
             W_o2, b_o2[None, :], Wsem8, bsem8, ncoef2, off2, P16, P8, P3)
    qe1 = _stage_c(egi1, egj1, DR, *cargs, 0, n1)
    qe2 = _stage_c(egi2, egj2, DR, *cargs, SPLIT, n2)
    s128a, s128b = _stage_d(qe1, qe2, idx_j, zeros_pad)
    s8 = (s128a + s128b)[:, 16:24]
    sg = _stage_f(s8, idx_j)
    pa = _stage_g(qe1, sg, U16, U8, Rep16, Rep7, W_mix128, 0, 0, 64000)
    pb = _stage_g(qe1, sg, U16, U8, Rep16, Rep7, W_mix128, 64000, 64000, 32000)
    pc = _stage_g(qe2, sg, U16, U8, Rep16, Rep7, W_mix128, 0, SPLIT, n2)
    ga = _stage_h(pa[0], pa[1], pa[2], pa[3], idx_j, zeros_pad, 0, 64000)
    gb = _stage_h(pb[0], pb[1], pb[2], pb[3], idx_j, zeros_pad, 64000, 32000)
    gc = _stage_h(pc[0], pc[1], pc[2], pc[3], idx_j, zeros_pad, SPLIT, n2)
    out = _stage_i(q, ga, gb, gc, s8, Wp1p, b_p1[None, :], W_p2,
                   b_p2[None, :], Wn1q, Wn1ij, Wn1c, b_n1[None, :], W_n2,
                   b_n2[None, :])
    return out


# speedup vs baseline: 1.0647x; 1.0402x over previous
import functools

import jax
import jax.numpy as jnp
from jax import lax
from jax.experimental import pallas as pl
from jax.experimental.pallas import tpu as pltpu
from jax.experimental.pallas import tpu_sc as plsc

N_ATOMS = 10000
N_PAIRS = 160000
NPAD = 10240
IN_F = 128
HID = 16
N_RBF = 43
N_HEADS = 7
N_COEF = 112
SPLIT = 96000

_EB = 2000
_NB = 1000


def _f32(*shape):
    return jax.ShapeDtypeStruct(shape, jnp.float32)


def _stage_a_body(q_ref, w_ref, tab_ref):
    tab_ref[...] = jnp.dot(q_ref[...], w_ref[...],
                           preferred_element_type=jnp.float32)


def _stage_a(q, WALL, *, interpret=False):
    grid = N_ATOMS // _NB
    return pl.pallas_call(
        _stage_a_body,
        grid=(grid,),
        in_specs=[
            pl.BlockSpec((_NB, IN_F), lambda i: (i, 0)),
            pl.BlockSpec((IN_F, 128), lambda i: (0, 0)),
        ],
        out_specs=pl.BlockSpec((_NB, 128), lambda i: (i, 0)),
        out_shape=_f32(N_ATOMS, 128),
        interpret=interpret,
    )(q, WALL)


def _stage_b(tab, idx_i, idx_j, e_lo, n_edges):
    mesh = plsc.VectorSubcoreMesh(core_axis_name="c", subcore_axis_name="s")
    nw = mesh.num_cores * mesh.num_subcores
    per_w = n_edges // nw
    chunk = 200
    n_chunks = per_w // chunk

    @functools.partial(
        pl.kernel,
        out_type=[_f32(n_edges, 128)] * 2,
        mesh=mesh,
        scratch_types=[
            pltpu.VMEM((chunk,), jnp.int32),
            pltpu.VMEM((chunk, 128), jnp.float32),
            pltpu.SemaphoreType.DMA,
        ],
    )
    def k(tab_hbm, ii_hbm, ij_hbm, oi_hbm, oj_hbm, idx_v, rows_v, sem):
        wid = lax.axis_index("s") * mesh.num_cores + lax.axis_index("c")
        base = wid * per_w
        for idx_hbm, out_hbm in ((ii_hbm, oi_hbm), (ij_hbm, oj_hbm)):
            for j in range(n_chunks):
                e0 = base + j * chunk
                pltpu.sync_copy(idx_hbm.at[pl.ds(e_lo + e0, chunk)], idx_v)
                pltpu.async_copy(tab_hbm.at[idx_v], rows_v, sem).wait()
                pltpu.sync_copy(rows_v, out_hbm.at[pl.ds(e0, chunk), :])

    return k(tab, idx_i, idx_j)


def _celu2(x):
    return jnp.where(x > 0, x, 2.0 * (jnp.exp(x * 0.5) - 1.0))


def _stage_c_body(egi_ref, egj_ref, dr_ref, m1i_ref, m1j_ref, m2i_ref, m2j_ref,
                  w43_ref, wd_ref, bin_ref, bo1_ref, wo2_ref, bo2_ref,
                  wsem_ref, bsem_ref, ncoef_ref, off_ref, p16_ref, p8_ref,
                  p3_ref, qe_ref):
    egi = egi_ref[...]
    egj = egj_ref[...]
    dr = dr_ref[...]
    d = dr[:, 0:1]
    q_in = (jnp.dot(egi, m1i_ref[...], preferred_element_type=jnp.float32)
            + jnp.dot(egj, m1j_ref[...], preferred_element_type=jnp.float32)
            + bin_ref[...])
    rbf = jnp.exp(ncoef_ref[...] * (d - off_ref[...]) ** 2)
    q_filt = rbf * q_in
    hpre = (jnp.dot(egi, m2i_ref[...], preferred_element_type=jnp.float32)
            + jnp.dot(egj, m2j_ref[...], preferred_element_type=jnp.float32)
            + jnp.dot(q_filt, w43_ref[...], preferred_element_type=jnp.float32)
            + d * wd_ref[...] + bo1_ref[...])
    h = jax.nn.silu(hpre)
    q16 = jnp.dot(h, wo2_ref[...], preferred_element_type=jnp.float32) + bo2_ref[...]
    att8 = jnp.dot(q16, wsem_ref[...], preferred_element_type=jnp.float32) + bsem_ref[...]
    e8 = jnp.exp(_celu2(att8))
    rhat = dr[:, 1:4] * (1.0 / (d + 1e-05))
    qe_ref[...] = (jnp.dot(q16, p16_ref[...], preferred_element_type=jnp.float32)
                   + jnp.dot(e8, p8_ref[...], preferred_element_type=jnp.float32)
                   + jnp.dot(rhat, p3_ref[...], preferred_element_type=jnp.float32))


def _stage_c(egi, egj, dr, M1i, M1j, M2i, M2j, W43, wd, b_in2, b_o12, W_o2,
             b_o22, Wsem8, bsem8, ncoef2, off2, P16, P8, P3, e_lo, n_edges,
             *, interpret=False):
    grid = n_edges // _EB
    off = e_lo // _EB
    full = lambda a, b: pl.BlockSpec((a, b), lambda i: (0, 0))
    return pl.pallas_call(
        _stage_c_body,
        grid=(grid,),
        in_specs=[
            pl.BlockSpec((_EB, 128), lambda i: (i, 0)),
            pl.BlockSpec((_EB, 128), lambda i: (i, 0)),
            pl.BlockSpec((_EB, 4), lambda i: (i + off, 0)),
            full(128, N_RBF), full(128, N_RBF), full(128, HID), full(128, HID),
            full(N_RBF, HID), full(1, HID),
            full(1, N_RBF), full(1, HID), full(HID, HID), full(1, HID),
            full(HID, 8), full(1, 8), full(1, N_RBF), full(1, N_RBF),
            full(HID, 128), full(8, 128), full(3, 128),
        ],
        out_specs=pl.BlockSpec((_EB, 128), lambda i: (i, 0)),
        out_shape=_f32(n_edges, 128),
        interpret=interpret,
    )(egi, egj, dr, M1i, M1j, M2i, M2j, W43, wd, b_in2, b_o12, W_o2, b_o22,
      Wsem8, bsem8, ncoef2, off2, P16, P8, P3)


def _stage_d(qe1, qe2, idx_j, zeros_pad):
    mesh = plsc.VectorSubcoreMesh(core_axis_name="c", subcore_axis_name="s")
    ns = mesh.num_subcores
    chunk = 200
    half1 = SPLIT // 2
    half2 = (N_PAIRS - SPLIT) // 2
    per_t1 = half1 // ns
    per_t2 = half2 // ns
    stripe = NPAD // ns

    @functools.partial(
        pl.kernel,
        out_type=[_f32(NPAD, 128)] * 2,
        mesh=mesh,
        scratch_types=[
            pltpu.VMEM((chunk,), jnp.int32),
            pltpu.VMEM((chunk, 128), jnp.float32),
            pltpu.VMEM_SHARED((NPAD, 128), jnp.float32),
        ],
    )
    def k(q1_hbm, q2_hbm, ij_hbm, z_hbm, s0_hbm, s1_hbm, idx_v, upd_v, acc):
        cid = lax.axis_index("c")
        sid = lax.axis_index("s")
        r0 = sid * stripe
        pltpu.sync_copy(z_hbm.at[pl.ds(r0, stripe), :], acc.at[pl.ds(r0, stripe)])
        plsc.subcore_barrier()
        for qe_hbm, g_base, half, per_t in ((q1_hbm, 0, half1, per_t1),
                                            (q2_hbm, SPLIT, half2, per_t2)):
            for j in range(per_t // chunk):
                e0 = cid * half + sid * per_t + j * chunk
                pltpu.sync_copy(ij_hbm.at[pl.ds(g_base + e0, chunk)], idx_v)
                pltpu.sync_copy(qe_hbm.at[pl.ds(e0, chunk), :], upd_v)
                pltpu.sync_copy(upd_v, acc.at[idx_v], add=True)
        plsc.subcore_barrier()

        @pl.when(cid == 0)
        def _():
            pltpu.sync_copy(acc.at[pl.ds(r0, stripe)], s0_hbm.at[pl.ds(r0, stripe)])

        @pl.when(cid == 1)
        def _():
            pltpu.sync_copy(acc.at[pl.ds(r0, stripe)], s1_hbm.at[pl.ds(r0, stripe)])

    return k(qe1, qe2, idx_j, zeros_pad)


def _stage_f(s8, idx_j):
    mesh = plsc.VectorSubcoreMesh(core_axis_name="c", subcore_axis_name="s")
    nw = mesh.num_cores * mesh.num_subcores
    per_w = N_PAIRS // nw
    chunk = 1000
    n_chunks = per_w // chunk

    @functools.partial(
        pl.kernel,
        out_type=_f32(N_PAIRS, 8),
        mesh=mesh,
        compiler_params=pltpu.CompilerParams(use_tc_tiling_on_sc=False),
        scratch_types=[
            pltpu.VMEM((chunk,), jnp.int32),
            pltpu.VMEM((chunk, 8), jnp.float32),
            pltpu.SemaphoreType.DMA,
        ],
    )
    def k(s_hbm, ij_hbm, out_hbm, idx_v, rows_v, sem):
        wid = lax.axis_index("s") * mesh.num_cores + lax.axis_index("c")
        base = wid * per_w
        for j in range(n_chunks):
            e0 = base + j * chunk
            pltpu.sync_copy(ij_hbm.at[pl.ds(e0, chunk)], idx_v)
            pltpu.async_copy(s_hbm.at[idx_v], rows_v, sem).wait()
            pltpu.sync_copy(rows_v, out_hbm.at[pl.ds(e0, chunk), :])

    return k(s8, idx_j)


def _stage_g_body(qe_ref, sg_ref, u16_ref, u8_ref, rep16_ref, rep7_ref,
                  wmix_ref, p0_ref, p1_ref, p2_ref, p3_ref):
    qe = qe_ref[...]
    q16 = jnp.dot(qe, u16_ref[...], preferred_element_type=jnp.float32)
    e8 = jnp.dot(qe, u8_ref[...], preferred_element_type=jnp.float32)
    comb = e8 / sg_ref[...]
    qr = jnp.dot(q16, rep16_ref[...], preferred_element_type=jnp.float32)
    cr = jnp.dot(comb, rep7_ref[...], preferred_element_type=jnp.float32)
    qia = qr * cr
    co = jnp.tanh(jnp.dot(qia, wmix_ref[...], preferred_element_type=jnp.float32))
    p0_ref[...] = qia
    p1_ref[...] = qe[:, 24:25] * co
    p2_ref[...] = qe[:, 25:26] * co
    p3_ref[...] = qe[:, 26:27] * co


def _stage_g(qe, sg, U16, U8, Rep16, Rep7, W_mix128,
             qe_lo, g_lo, n_edges, *, interpret=False):
    grid = n_edges // _EB
    qoff = qe_lo // _EB
    goff = g_lo // _EB
    full = lambda a, b: pl.BlockSpec((a, b), lambda i: (0, 0))
    loc = lambda w: pl.BlockSpec((_EB, w), lambda i: (i + qoff, 0))
    glo = lambda w: pl.BlockSpec((_EB, w), lambda i: (i + goff, 0))
    return pl.pallas_call(
        _stage_g_body,
        grid=(grid,),
        in_specs=[
            loc(128), glo(8),
            full(128, HID), full(128, 8),
            full(HID, 128), full(8, 128), full(128, 128),
        ],
        out_specs=[pl.BlockSpec((_EB, 128), lambda i: (i, 0))] * 4,
        out_shape=[_f32(n_edges, 128)] * 4,
        interpret=interpret,
    )(qe, sg, U16, U8, Rep16, Rep7, W_mix128)


def _stage_h(pay0, pay1, pay2, pay3, idx_j, zeros_pad, e_lo, n_edges):
    mesh = plsc.VectorSubcoreMesh(core_axis_name="c", subcore_axis_name="s")
    ns = mesh.num_subcores
    per_t = n_edges // ns
    chunk = 200
    n_chunks = per_t // chunk
    stripe = NPAD // ns

    @functools.partial(
        pl.kernel,
        out_type=[_f32(NPAD, 128)] * 4,
        mesh=mesh,
        scratch_types=[
            pltpu.VMEM((chunk,), jnp.int32),
            pltpu.VMEM((chunk, 128), jnp.float32),
            pltpu.VMEM_SHARED((NPAD, 128), jnp.float32),
        ],
    )
    def k(p0_hbm, p1_hbm, p2_hbm, p3_hbm, ij_hbm, z_hbm,
          o0_hbm, o1_hbm, o2_hbm, o3_hbm, idx_v, upd_v, acc):
        cid = lax.axis_index("c")
        sid = lax.axis_index("s")
        r0 = sid * stripe

        def run_group(pay_hbm, out_hbm):
            pltpu.sync_copy(z_hbm.at[pl.ds(r0, stripe), :], acc.at[pl.ds(r0, stripe)])
            plsc.subcore_barrier()
            for j in range(n_chunks):
                e0 = sid * per_t + j * chunk
                pltpu.sync_copy(ij_hbm.at[pl.ds(e_lo + e0, chunk)], idx_v)
                pltpu.sync_copy(pay_hbm.at[pl.ds(e0, chunk), :], upd_v)
                pltpu.sync_copy(upd_v, acc.at[idx_v], add=True)
            plsc.subcore_barrier()
            pltpu.sync_copy(acc.at[pl.ds(r0, stripe)], out_hbm.at[pl.ds(r0, stripe)])
            plsc.subcore_barrier()

        @pl.when(cid == 0)
        def _():
            run_group(p0_hbm, o0_hbm)
            run_group(p1_hbm, o1_hbm)

        @pl.when(cid == 1)
        def _():
            run_group(p2_hbm, o2_hbm)
            run_group(p3_hbm, o3_hbm)

    return k(pay0, pay1, pay2, pay3, idx_j, zeros_pad)


def _stage_i_body(q_ref, g0a_ref, g1a_ref, g2a_ref, g3a_ref,
                  g0b_ref, g1b_ref, g2b_ref, g3b_ref, s_ref,
                  wp1_ref, bp1_ref, wp2_ref, bp2_ref,
                  wn1q_ref, wn1ij_ref, wn1c_ref, bn1_ref, wn2_ref, bn2_ref,
                  out_ref):
    q = q_ref[...]
    qij = g0a_ref[...] + g0b_ref[...]
    inv = 1.0 / jnp.maximum(s_ref[...][:, 7:8], 1.0)
    m1 = (g1a_ref[...] + g1b_ref[...]) * inv
    m2 = (g2a_ref[...] + g2b_ref[...]) * inv
    m3 = (g3a_ref[...] + g3b_ref[...]) * inv
    comb_norm = m1 * m1 + m2 * m2 + m3 * m3
    qc = jax.nn.silu(jnp.dot(comb_norm, wp1_ref[...],
                             preferred_element_type=jnp.float32) + bp1_ref[...])
    qc = jax.nn.silu(jnp.dot(qc, wp2_ref[...],
                             preferred_element_type=jnp.float32) + bp2_ref[...])
    o = (jnp.dot(q, wn1q_ref[...], preferred_element_type=jnp.float32)
         + jnp.dot(qij, wn1ij_ref[...], preferred_element_type=jnp.float32)
         + jnp.dot(qc, wn1c_ref[...], preferred_element_type=jnp.float32)
         + bn1_ref[...])
    o = jax.nn.silu(o)
    o = jax.nn.silu(jnp.dot(o, wn2_ref[...], preferred_element_type=jnp.float32)
                    + bn2_ref[...])
    out_ref[...] = q + o


def _stage_i(q, ga, gb, s8, W_p1p, b_p12, W_p2, b_p22,
             Wn1q, Wn1ij, Wn1c, b_n12, W_n2, b_n22, *, interpret=False):
    grid = N_ATOMS // _NB
    full = lambda a, b: pl.BlockSpec((a, b), lambda i: (0, 0))
    nb = lambda w: pl.BlockSpec((_NB, w), lambda i: (i, 0))
    return pl.pallas_call(
        _stage_i_body,
        grid=(grid,),
        in_specs=[
            nb(IN_F), nb(128), nb(128), nb(128), nb(128),
            nb(128), nb(128), nb(128), nb(128), nb(8),
            full(128, HID), full(1, HID), full(HID, HID), full(1, HID),
            full(IN_F, HID), full(128, HID), full(HID, HID), full(1, HID),
            full(HID, IN_F), full(1, IN_F),
        ],
        out_specs=nb(IN_F),
        out_shape=_f32(N_ATOMS, IN_F),
        interpret=interpret,
    )(q, ga[0], ga[1], ga[2], ga[3], gb[0], gb[1], gb[2], gb[3], s8,
      W_p1p, b_p12, W_p2, b_p22, Wn1q, Wn1ij, Wn1c, b_n12, W_n2, b_n22)


def kernel(q, mu, r_ij, d_ij, idx_i, idx_j, rbf_offsets, rbf_widths, W_in, b_in,
           W_o1, b_o1, W_o2, b_o2, W_sem, b_sem, W_mix, W_p1, b_p1, W_p2, b_p2,
           W_n1, b_n1, W_n2, b_n2):
    f32 = jnp.float32
    z5 = jnp.zeros((IN_F, 5), f32)
    WALL = jnp.concatenate([W_in[:IN_F], z5, W_o1[:IN_F],
                            W_in[IN_F:], z5, W_o1[IN_F:2 * IN_F]], axis=1)
    eye43 = jnp.eye(N_RBF, dtype=f32)
    eye16 = jnp.eye(HID, dtype=f32)
    eye8 = jnp.eye(8, dtype=f32)
    M1i = jnp.concatenate([eye43, jnp.zeros((85, N_RBF), f32)], axis=0)
    M1j = jnp.concatenate([jnp.zeros((64, N_RBF), f32), eye43,
                           jnp.zeros((21, N_RBF), f32)], axis=0)
    M2i = jnp.concatenate([jnp.zeros((48, HID), f32), eye16,
                           jnp.zeros((64, HID), f32)], axis=0)
    M2j = jnp.concatenate([jnp.zeros((112, HID), f32), eye16], axis=0)
    W43 = W_o1[2 * IN_F:2 * IN_F + N_RBF]
    wd = W_o1[2 * IN_F + N_RBF:2 * IN_F + N_RBF + 1]
    Wsem8 = jnp.concatenate([W_sem, jnp.zeros((HID, 1), f32)], axis=1)
    bsem8 = jnp.concatenate([b_sem, jnp.zeros((1,), f32)])[None, :]
    ncoef2 = (-0.5 / (rbf_widths ** 2))[None, :]
    off2 = rbf_offsets[None, :]
    P16 = jnp.concatenate([eye16, jnp.zeros((HID, 112), f32)], axis=1)
    P8 = jnp.concatenate([jnp.zeros((8, 16), f32), eye8,
                          jnp.zeros((8, 104), f32)], axis=1)
    U16 = P16.T
    U8 = P8.T
    ar = jnp.arange(128)
    Rep16 = ((ar[None, :] // N_HEADS == jnp.arange(HID)[:, None])
             & (ar[None, :] < N_COEF)).astype(f32)
    Rep7 = ((ar[None, :] % N_HEADS == jnp.arange(8)[:, None])
            & (ar[None, :] < N_COEF)).astype(f32)
    W_mix128 = jnp.zeros((128, 128), f32).at[:N_COEF, :N_COEF].set(W_mix)
    P3 = jnp.zeros((3, 128), f32).at[:, 24:27].set(jnp.eye(3, dtype=f32))
    Wn1q = W_n1[:IN_F]
    Wn1ij = jnp.concatenate([W_n1[IN_F:IN_F + N_COEF],
                             jnp.zeros((16, HID), f32)], axis=0)
    Wn1c = W_n1[IN_F + N_COEF:]
    Wp1p = jnp.concatenate([W_p1, jnp.zeros((16, HID), f32)], axis=0)
    zeros_pad = jnp.zeros((NPAD, 128), f32)
    DR = jnp.concatenate([d_ij[:, None], r_ij], axis=1)

    n1, n2 = SPLIT, N_PAIRS - SPLIT

    tab = _stage_a(q, WALL)
    egi1, egj1 = _stage_b(tab, idx_i, idx_j, 0, n1)
    egi2, egj2 = _stage_b(tab, idx_i, idx_j, SPLIT, n2)
    cargs = (M1i, M1j, M2i, M2j, W43, wd, b_in[None, :], b_o1[None, :],
             W_o2, b_o2[None, :], Wsem8, bsem8, ncoef2, off2, P16, P8, P3)
    qe1 = _stage_c(egi1, egj1, DR, *cargs, 0, n1)
    qe2 = _stage_c(egi2, egj2, DR, *cargs, SPLIT, n2)
    s128a, s128b = _stage_d(qe1, qe2, idx_j, zeros_pad)
    s8 = (s128a + s128b)[:, 16:24]
    sg = _stage_f(s8, idx_j)
    pa = _stage_g(qe1, sg, U16, U8, Rep16, Rep7, W_mix128, 0, 0, n1)
    pc = _stage_g(qe2, sg, U16, U8, Rep16, Rep7, W_mix128, 0, SPLIT, n2)
    ga = _stage_h(pa[0], pa[1], pa[2], pa[3], idx_j, zeros_pad, 0, n1)
    gc = _stage_h(pc[0], pc[1], pc[2], pc[3], idx_j, zeros_pad, SPLIT, n2)
    out = _stage_i(q, ga, gc, s8, Wp1p, b_p1[None, :], W_p2,
                   b_p2[None, :], Wn1q, Wn1ij, Wn1c, b_n1[None, :], W_n2,
                   b_n2[None, :])
    return out


# speedup vs baseline: 1.0741x; 1.0088x over previous
import functools

import jax
import jax.numpy as jnp
from jax import lax
from jax.experimental import pallas as pl
from jax.experimental.pallas import tpu as pltpu
from jax.experimental.pallas import tpu_sc as plsc

N_ATOMS = 10000
N_PAIRS = 160000
NPAD = 10240
IN_F = 128
HID = 16
N_RBF = 43
N_HEADS = 7
N_COEF = 112
SPLIT = 96000

_EB = 2000
_NB = 1000


def _f32(*shape):
    return jax.ShapeDtypeStruct(shape, jnp.float32)


def _stage_a_body(q_ref, w_ref, tab_ref):
    tab_ref[...] = jnp.dot(q_ref[...], w_ref[...],
                           preferred_element_type=jnp.float32)


def _stage_a(q, WALL, *, interpret=False):
    grid = N_ATOMS // _NB
    return pl.pallas_call(
        _stage_a_body,
        grid=(grid,),
        in_specs=[
            pl.BlockSpec((_NB, IN_F), lambda i: (i, 0)),
            pl.BlockSpec((IN_F, 128), lambda i: (0, 0)),
        ],
        out_specs=pl.BlockSpec((_NB, 128), lambda i: (i, 0)),
        out_shape=_f32(N_ATOMS, 128),
        interpret=interpret,
    )(q, WALL)


def _stage_b(tab, idx_i, idx_j, e_lo, n_edges):
    mesh = plsc.VectorSubcoreMesh(core_axis_name="c", subcore_axis_name="s")
    nw = mesh.num_cores * mesh.num_subcores
    per_w = n_edges // nw
    chunk = 1000
    n_chunks = per_w // chunk

    @functools.partial(
        pl.kernel,
        out_type=[_f32(n_edges, 128)] * 2,
        mesh=mesh,
        scratch_types=[
            pltpu.VMEM((chunk,), jnp.int32),
            pltpu.VMEM((chunk, 128), jnp.float32),
            pltpu.SemaphoreType.DMA,
        ],
    )
    def k(tab_hbm, ii_hbm, ij_hbm, oi_hbm, oj_hbm, idx_v, rows_v, sem):
        wid = lax.axis_index("s") * mesh.num_cores + lax.axis_index("c")
        base = wid * per_w
        for idx_hbm, out_hbm in ((ii_hbm, oi_hbm), (ij_hbm, oj_hbm)):
            for j in range(n_chunks):
                e0 = base + j * chunk
                pltpu.sync_copy(idx_hbm.at[pl.ds(e_lo + e0, chunk)], idx_v)
                pltpu.async_copy(tab_hbm.at[idx_v], rows_v, sem).wait()
                pltpu.sync_copy(rows_v, out_hbm.at[pl.ds(e0, chunk), :])

    return k(tab, idx_i, idx_j)


def _celu2(x):
    return jnp.where(x > 0, x, 2.0 * (jnp.exp(x * 0.5) - 1.0))


def _stage_c_body(egi_ref, egj_ref, dr_ref, m1i_ref, m1j_ref, m2i_ref, m2j_ref,
                  w43_ref, wd_ref, bin_ref, bo1_ref, wo2_ref, bo2_ref,
                  wsem_ref, bsem_ref, ncoef_ref, off_ref, p16_ref, p8_ref,
                  p3_ref, qe_ref):
    egi = egi_ref[...]
    egj = egj_ref[...]
    dr = dr_ref[...]
    d = dr[:, 0:1]
    q_in = (jnp.dot(egi, m1i_ref[...], preferred_element_type=jnp.float32)
            + jnp.dot(egj, m1j_ref[...], preferred_element_type=jnp.float32)
            + bin_ref[...])
    rbf = jnp.exp(ncoef_ref[...] * (d - off_ref[...]) ** 2)
    q_filt = rbf * q_in
    hpre = (jnp.dot(egi, m2i_ref[...], preferred_element_type=jnp.float32)
            + jnp.dot(egj, m2j_ref[...], preferred_element_type=jnp.float32)
            + jnp.dot(q_filt, w43_ref[...], preferred_element_type=jnp.float32)
            + d * wd_ref[...] + bo1_ref[...])
    h = jax.nn.silu(hpre)
    q16 = jnp.dot(h, wo2_ref[...], preferred_element_type=jnp.float32) + bo2_ref[...]
    att8 = jnp.dot(q16, wsem_ref[...], preferred_element_type=jnp.float32) + bsem_ref[...]
    e8 = jnp.exp(_celu2(att8))
    rhat = dr[:, 1:4] * (1.0 / (d + 1e-05))
    qe_ref[...] = (jnp.dot(q16, p16_ref[...], preferred_element_type=jnp.float32)
                   + jnp.dot(e8, p8_ref[...], preferred_element_type=jnp.float32)
                   + jnp.dot(rhat, p3_ref[...], preferred_element_type=jnp.float32))


def _stage_c(egi, egj, dr, M1i, M1j, M2i, M2j, W43, wd, b_in2, b_o12, W_o2,
             b_o22, Wsem8, bsem8, ncoef2, off2, P16, P8, P3, e_lo, n_edges,
             *, interpret=False):
    grid = n_edges // _EB
    off = e_lo // _EB
    full = lambda a, b: pl.BlockSpec((a, b), lambda i: (0, 0))
    return pl.pallas_call(
        _stage_c_body,
        grid=(grid,),
        in_specs=[
            pl.BlockSpec((_EB, 128), lambda i: (i, 0)),
            pl.BlockSpec((_EB, 128), lambda i: (i, 0)),
            pl.BlockSpec((_EB, 4), lambda i: (i + off, 0)),
            full(128, N_RBF), full(128, N_RBF), full(128, HID), full(128, HID),
            full(N_RBF, HID), full(1, HID),
            full(1, N_RBF), full(1, HID), full(HID, HID), full(1, HID),
            full(HID, 8), full(1, 8), full(1, N_RBF), full(1, N_RBF),
            full(HID, 128), full(8, 128), full(3, 128),
        ],
        out_specs=pl.BlockSpec((_EB, 128), lambda i: (i, 0)),
        out_shape=_f32(n_edges, 128),
        interpret=interpret,
    )(egi, egj, dr, M1i, M1j, M2i, M2j, W43, wd, b_in2, b_o12, W_o2, b_o22,
      Wsem8, bsem8, ncoef2, off2, P16, P8, P3)


def _stage_d(qe1, qe2, idx_j, zeros_pad):
    mesh = plsc.VectorSubcoreMesh(core_axis_name="c", subcore_axis_name="s")
    ns = mesh.num_subcores
    chunk = 200
    half1 = SPLIT // 2
    half2 = (N_PAIRS - SPLIT) // 2
    per_t1 = half1 // ns
    per_t2 = half2 // ns
    stripe = NPAD // ns

    @functools.partial(
        pl.kernel,
        out_type=[_f32(NPAD, 128)] * 2,
        mesh=mesh,
        scratch_types=[
            pltpu.VMEM((chunk,), jnp.int32),
            pltpu.VMEM((chunk, 128), jnp.float32),
            pltpu.VMEM_SHARED((NPAD, 128), jnp.float32),
        ],
    )
    def k(q1_hbm, q2_hbm, ij_hbm, z_hbm, s0_hbm, s1_hbm, idx_v, upd_v, acc):
        cid = lax.axis_index("c")
        sid = lax.axis_index("s")
        r0 = sid * stripe
        pltpu.sync_copy(z_hbm.at[pl.ds(r0, stripe), :], acc.at[pl.ds(r0, stripe)])
        plsc.subcore_barrier()
        for qe_hbm, g_base, half, per_t in ((q1_hbm, 0, half1, per_t1),
                                            (q2_hbm, SPLIT, half2, per_t2)):
            for j in range(per_t // chunk):
                e0 = cid * half + sid * per_t + j * chunk
                pltpu.sync_copy(ij_hbm.at[pl.ds(g_base + e0, chunk)], idx_v)
                pltpu.sync_copy(qe_hbm.at[pl.ds(e0, chunk), :], upd_v)
                pltpu.sync_copy(upd_v, acc.at[idx_v], add=True)
        plsc.subcore_barrier()

        @pl.when(cid == 0)
        def _():
            pltpu.sync_copy(acc.at[pl.ds(r0, stripe)], s0_hbm.at[pl.ds(r0, stripe)])

        @pl.when(cid == 1)
        def _():
            pltpu.sync_copy(acc.at[pl.ds(r0, stripe)], s1_hbm.at[pl.ds(r0, stripe)])

    return k(qe1, qe2, idx_j, zeros_pad)


def _stage_f(s8, idx_j):
    mesh = plsc.VectorSubcoreMesh(core_axis_name="c", subcore_axis_name="s")
    nw = mesh.num_cores * mesh.num_subcores
    per_w = N_PAIRS // nw
    chunk = 1000
    n_chunks = per_w // chunk

    @functools.partial(
        pl.kernel,
        out_type=_f32(N_PAIRS, 8),
        mesh=mesh,
        compiler_params=pltpu.CompilerParams(use_tc_tiling_on_sc=False),
        scratch_types=[
            pltpu.VMEM((chunk,), jnp.int32),
            pltpu.VMEM((chunk, 8), jnp.float32),
            pltpu.SemaphoreType.DMA,
        ],
    )
    def k(s_hbm, ij_hbm, out_hbm, idx_v, rows_v, sem):
        wid = lax.axis_index("s") * mesh.num_cores + lax.axis_index("c")
        base = wid * per_w
        for j in range(n_chunks):
            e0 = base + j * chunk
            pltpu.sync_copy(ij_hbm.at[pl.ds(e0, chunk)], idx_v)
            pltpu.async_copy(s_hbm.at[idx_v], rows_v, sem).wait()
            pltpu.sync_copy(rows_v, out_hbm.at[pl.ds(e0, chunk), :])

    return k(s8, idx_j)


def _stage_g_body(qe_ref, sg_ref, u16_ref, u8_ref, rep16_ref, rep7_ref,
                  wmix_ref, p0_ref, p1_ref, p2_ref, p3_ref):
    qe = qe_ref[...]
    q16 = jnp.dot(qe, u16_ref[...], preferred_element_type=jnp.float32)
    e8 = jnp.dot(qe, u8_ref[...], preferred_element_type=jnp.float32)
    comb = e8 / sg_ref[...]
    qr = jnp.dot(q16, rep16_ref[...], preferred_element_type=jnp.float32)
    cr = jnp.dot(comb, rep7_ref[...], preferred_element_type=jnp.float32)
    qia = qr * cr
    co = jnp.tanh(jnp.dot(qia, wmix_ref[...], preferred_element_type=jnp.float32))
    p0_ref[...] = qia
    p1_ref[...] = qe[:, 24:25] * co
    p2_ref[...] = qe[:, 25:26] * co
    p3_ref[...] = qe[:, 26:27] * co


def _stage_g(qe, sg, U16, U8, Rep16, Rep7, W_mix128,
             qe_lo, g_lo, n_edges, *, interpret=False):
    grid = n_edges // _EB
    qoff = qe_lo // _EB
    goff = g_lo // _EB
    full = lambda a, b: pl.BlockSpec((a, b), lambda i: (0, 0))
    loc = lambda w: pl.BlockSpec((_EB, w), lambda i: (i + qoff, 0))
    glo = lambda w: pl.BlockSpec((_EB, w), lambda i: (i + goff, 0))
    return pl.pallas_call(
        _stage_g_body,
        grid=(grid,),
        in_specs=[
            loc(128), glo(8),
            full(128, HID), full(128, 8),
            full(HID, 128), full(8, 128), full(128, 128),
        ],
        out_specs=[pl.BlockSpec((_EB, 128), lambda i: (i, 0))] * 4,
        out_shape=[_f32(n_edges, 128)] * 4,
        interpret=interpret,
    )(qe, sg, U16, U8, Rep16, Rep7, W_mix128)


def _stage_h(pay0, pay1, pay2, pay3, idx_j, zeros_pad, e_lo, n_edges):
    mesh = plsc.VectorSubcoreMesh(core_axis_name="c", subcore_axis_name="s")
    ns = mesh.num_subcores
    per_t = n_edges // ns
    chunk = 200
    n_chunks = per_t // chunk
    stripe = NPAD // ns

    @functools.partial(
        pl.kernel,
        out_type=[_f32(NPAD, 128)] * 4,
        mesh=mesh,
        scratch_types=[
            pltpu.VMEM((chunk,), jnp.int32),
            pltpu.VMEM((chunk, 128), jnp.float32),
            pltpu.VMEM_SHARED((NPAD, 128), jnp.float32),
        ],
    )
    def k(p0_hbm, p1_hbm, p2_hbm, p3_hbm, ij_hbm, z_hbm,
          o0_hbm, o1_hbm, o2_hbm, o3_hbm, idx_v, upd_v, acc):
        cid = lax.axis_index("c")
        sid = lax.axis_index("s")
        r0 = sid * stripe

        def run_group(pay_hbm, out_hbm):
            pltpu.sync_copy(z_hbm.at[pl.ds(r0, stripe), :], acc.at[pl.ds(r0, stripe)])
            plsc.subcore_barrier()
            for j in range(n_chunks):
                e0 = sid * per_t + j * chunk
                pltpu.sync_copy(ij_hbm.at[pl.ds(e_lo + e0, chunk)], idx_v)
                pltpu.sync_copy(pay_hbm.at[pl.ds(e0, chunk), :], upd_v)
                pltpu.sync_copy(upd_v, acc.at[idx_v], add=True)
            plsc.subcore_barrier()
            pltpu.sync_copy(acc.at[pl.ds(r0, stripe)], out_hbm.at[pl.ds(r0, stripe)])
            plsc.subcore_barrier()

        @pl.when(cid == 0)
        def _():
            run_group(p0_hbm, o0_hbm)
            run_group(p1_hbm, o1_hbm)

        @pl.when(cid == 1)
        def _():
            run_group(p2_hbm, o2_hbm)
            run_group(p3_hbm, o3_hbm)

    return k(pay0, pay1, pay2, pay3, idx_j, zeros_pad)


def _stage_i_body(q_ref, g0a_ref, g1a_ref, g2a_ref, g3a_ref,
                  g0b_ref, g1b_ref, g2b_ref, g3b_ref, s_ref,
                  wp1_ref, bp1_ref, wp2_ref, bp2_ref,
                  wn1q_ref, wn1ij_ref, wn1c_ref, bn1_ref, wn2_ref, bn2_ref,
                  out_ref):
    q = q_ref[...]
    qij = g0a_ref[...] + g0b_ref[...]
    inv = 1.0 / jnp.maximum(s_ref[...][:, 7:8], 1.0)
    m1 = (g1a_ref[...] + g1b_ref[...]) * inv
    m2 = (g2a_ref[...] + g2b_ref[...]) * inv
    m3 = (g3a_ref[...] + g3b_ref[...]) * inv
    comb_norm = m1 * m1 + m2 * m2 + m3 * m3
    qc = jax.nn.silu(jnp.dot(comb_norm, wp1_ref[...],
                             preferred_element_type=jnp.float32) + bp1_ref[...])
    qc = jax.nn.silu(jnp.dot(qc, wp2_ref[...],
                             preferred_element_type=jnp.float32) + bp2_ref[...])
    o = (jnp.dot(q, wn1q_ref[...], preferred_element_type=jnp.float32)
         + jnp.dot(qij, wn1ij_ref[...], preferred_element_type=jnp.float32)
         + jnp.dot(qc, wn1c_ref[...], preferred_element_type=jnp.float32)
         + bn1_ref[...])
    o = jax.nn.silu(o)
    o = jax.nn.silu(jnp.dot(o, wn2_ref[...], preferred_element_type=jnp.float32)
                    + bn2_ref[...])
    out_ref[...] = q + o


def _stage_i(q, ga, gb, s8, W_p1p, b_p12, W_p2, b_p22,
             Wn1q, Wn1ij, Wn1c, b_n12, W_n2, b_n22, *, interpret=False):
    grid = N_ATOMS // _NB
    full = lambda a, b: pl.BlockSpec((a, b), lambda i: (0, 0))
    nb = lambda w: pl.BlockSpec((_NB, w), lambda i: (i, 0))
    return pl.pallas_call(
        _stage_i_body,
        grid=(grid,),
        in_specs=[
            nb(IN_F), nb(128), nb(128), nb(128), nb(128),
            nb(128), nb(128), nb(128), nb(128), nb(8),
            full(128, HID), full(1, HID), full(HID, HID), full(1, HID),
            full(IN_F, HID), full(128, HID), full(HID, HID), full(1, HID),
            full(HID, IN_F), full(1, IN_F),
        ],
        out_specs=nb(IN_F),
        out_shape=_f32(N_ATOMS, IN_F),
        interpret=interpret,
    )(q, ga[0], ga[1], ga[2], ga[3], gb[0], gb[1], gb[2], gb[3], s8,
      W_p1p, b_p12, W_p2, b_p22, Wn1q, Wn1ij, Wn1c, b_n12, W_n2, b_n22)


def kernel(q, mu, r_ij, d_ij, idx_i, idx_j, rbf_offsets, rbf_widths, W_in, b_in,
           W_o1, b_o1, W_o2, b_o2, W_sem, b_sem, W_mix, W_p1, b_p1, W_p2, b_p2,
           W_n1, b_n1, W_n2, b_n2):
    f32 = jnp.float32
    z5 = jnp.zeros((IN_F, 5), f32)
    WALL = jnp.concatenate([W_in[:IN_F], z5, W_o1[:IN_F],
                            W_in[IN_F:], z5, W_o1[IN_F:2 * IN_F]], axis=1)
    eye43 = jnp.eye(N_RBF, dtype=f32)
    eye16 = jnp.eye(HID, dtype=f32)
    eye8 = jnp.eye(8, dtype=f32)
    M1i = jnp.concatenate([eye43, jnp.zeros((85, N_RBF), f32)], axis=0)
    M1j = jnp.concatenate([jnp.zeros((64, N_RBF), f32), eye43,
                           jnp.zeros((21, N_RBF), f32)], axis=0)
    M2i = jnp.concatenate([jnp.zeros((48, HID), f32), eye16,
                           jnp.zeros((64, HID), f32)], axis=0)
    M2j = jnp.concatenate([jnp.zeros((112, HID), f32), eye16], axis=0)
    W43 = W_o1[2 * IN_F:2 * IN_F + N_RBF]
    wd = W_o1[2 * IN_F + N_RBF:2 * IN_F + N_RBF + 1]
    Wsem8 = jnp.concatenate([W_sem, jnp.zeros((HID, 1), f32)], axis=1)
    bsem8 = jnp.concatenate([b_sem, jnp.zeros((1,), f32)])[None, :]
    ncoef2 = (-0.5 / (rbf_widths ** 2))[None, :]
    off2 = rbf_offsets[None, :]
    P16 = jnp.concatenate([eye16, jnp.zeros((HID, 112), f32)], axis=1)
    P8 = jnp.concatenate([jnp.zeros((8, 16), f32), eye8,
                          jnp.zeros((8, 104), f32)], axis=1)
    U16 = P16.T
    U8 = P8.T
    ar = jnp.arange(128)
    Rep16 = ((ar[None, :] // N_HEADS == jnp.arange(HID)[:, None])
             & (ar[None, :] < N_COEF)).astype(f32)
    Rep7 = ((ar[None, :] % N_HEADS == jnp.arange(8)[:, None])
            & (ar[None, :] < N_COEF)).astype(f32)
    W_mix128 = jnp.zeros((128, 128), f32).at[:N_COEF, :N_COEF].set(W_mix)
    P3 = jnp.zeros((3, 128), f32).at[:, 24:27].set(jnp.eye(3, dtype=f32))
    Wn1q = W_n1[:IN_F]
    Wn1ij = jnp.concatenate([W_n1[IN_F:IN_F + N_COEF],
                             jnp.zeros((16, HID), f32)], axis=0)
    Wn1c = W_n1[IN_F + N_COEF:]
    Wp1p = jnp.concatenate([W_p1, jnp.zeros((16, HID), f32)], axis=0)
    zeros_pad = jnp.zeros((NPAD, 128), f32)
    DR = jnp.concatenate([d_ij[:, None], r_ij], axis=1)

    n1, n2 = SPLIT, N_PAIRS - SPLIT

    tab = _stage_a(q, WALL)
    egi1, egj1 = _stage_b(tab, idx_i, idx_j, 0, n1)
    egi2, egj2 = _stage_b(tab, idx_i, idx_j, SPLIT, n2)
    cargs = (M1i, M1j, M2i, M2j, W43, wd, b_in[None, :], b_o1[None, :],
             W_o2, b_o2[None, :], Wsem8, bsem8, ncoef2, off2, P16, P8, P3)
    qe1 = _stage_c(egi1, egj1, DR, *cargs, 0, n1)
    qe2 = _stage_c(egi2, egj2, DR, *cargs, SPLIT, n2)
    s128a, s128b = _stage_d(qe1, qe2, idx_j, zeros_pad)
    s8 = (s128a + s128b)[:, 16:24]
    sg = _stage_f(s8, idx_j)
    pa = _stage_g(qe1, sg, U16, U8, Rep16, Rep7, W_mix128, 0, 0, n1)
    pc = _stage_g(qe2, sg, U16, U8, Rep16, Rep7, W_mix128, 0, SPLIT, n2)
    ga = _stage_h(pa[0], pa[1], pa[2], pa[3], idx_j, zeros_pad, 0, n1)
    gc = _stage_h(pc[0], pc[1], pc[2], pc[3], idx_j, zeros_pad, SPLIT, n2)
    out = _stage_i(q, ga, gc, s8, Wp1p, b_p1[None, :], W_p2,
                   b_p2[None, :], Wn1q, Wn1ij, Wn1c, b_n1[None, :], W_n2,
                   b_n2[None, :])
    return out
